# baseline (speedup 1.0000x reference)
import functools

import jax
import jax.numpy as jnp
from jax import lax
from jax.experimental import pallas as pl
from jax.experimental.pallas import tpu as pltpu
from jax.experimental.pallas import tpu_sc as plsc

N_POSES = 128
MAX_BLOCKS = 512
ATOMS_PER_BLOCK = 16
MAX_ATOMS = MAX_BLOCKS * ATOMS_PER_BLOCK
N_BT = 100
MAX_CONNS = 3

_N_TILES = 32
_POSES_PER_TILE = N_POSES // _N_TILES
_LANES = 16
_CHUNKS = MAX_BLOCKS // _LANES
_PACK_ROWS = 19


def _sc_gather(coords_t, offs, bts, iconns_t, tbl, pose_base, n_poses):
    poses_per_tile = n_poses // _N_TILES
    mesh = plsc.VectorSubcoreMesh(core_axis_name="c", subcore_axis_name="s")

    @functools.partial(
        pl.kernel,
        mesh=mesh,
        compiler_params=pltpu.CompilerParams(needs_layout_passes=False),
        out_type=jax.ShapeDtypeStruct((_PACK_ROWS, n_poses, MAX_BLOCKS), jnp.float32),
        scratch_types=[
            pltpu.VMEM((MAX_ATOMS,), jnp.float32),
            pltpu.VMEM((MAX_ATOMS,), jnp.float32),
            pltpu.VMEM((MAX_ATOMS,), jnp.float32),
            pltpu.VMEM((MAX_ATOMS,), jnp.float32),
            pltpu.VMEM((MAX_ATOMS,), jnp.float32),
            pltpu.VMEM((MAX_ATOMS,), jnp.float32),
            pltpu.VMEM((MAX_BLOCKS,), jnp.int32),
            pltpu.VMEM((MAX_BLOCKS,), jnp.int32),
            pltpu.VMEM((MAX_BLOCKS,), jnp.int32),
            pltpu.VMEM((MAX_BLOCKS,), jnp.int32),
            pltpu.VMEM((MAX_CONNS, 2, MAX_BLOCKS), jnp.int32),
            pltpu.VMEM((MAX_CONNS, 2, MAX_BLOCKS), jnp.int32),
            pltpu.VMEM((_PACK_ROWS, MAX_BLOCKS), jnp.float32),
            pltpu.VMEM((_PACK_ROWS, MAX_BLOCKS), jnp.float32),
            pltpu.VMEM((512,), jnp.int32),
            pltpu.VMEM((MAX_BLOCKS,), jnp.int32),
            pltpu.VMEM((MAX_BLOCKS,), jnp.int32),
            pltpu.SemaphoreType.DMA,
            pltpu.SemaphoreType.DMA,
            pltpu.SemaphoreType.DMA,
            pltpu.SemaphoreType.DMA,
        ],
    )
    def k(coords_hbm, offs_hbm, bts_hbm, iconns_hbm, tbl_hbm,
          out_hbm, cx0, cy0, cz0, cx1, cy1, cz1, offs0, bts0, offs1, bts1,
          ic0, ic1, o0v, o1v, tbl_v, s1_v, s2_v, sin0, sin1, sout0, sout1):
        wid = lax.axis_index("s") * 2 + lax.axis_index("c")
        bufs = [(cx0, cy0, cz0, offs0, bts0, ic0, o0v, sin0, sout0),
                (cx1, cy1, cz1, offs1, bts1, ic1, o1v, sin1, sout1)]
        pltpu.sync_copy(tbl_hbm, tbl_v)

        def stage(j):
            cx, cy, cz, off_v, bt_v, ic_v, _, sem, _2 = bufs[j & 1]
            p = pose_base + wid * poses_per_tile + j
            return [pltpu.async_copy(coords_hbm.at[0, p], cx, sem),
                    pltpu.async_copy(coords_hbm.at[1, p], cy, sem),
                    pltpu.async_copy(coords_hbm.at[2, p], cz, sem),
                    pltpu.async_copy(offs_hbm.at[p], off_v, sem),
                    pltpu.async_copy(bts_hbm.at[p], bt_v, sem),
                    pltpu.async_copy(iconns_hbm.at[p], ic_v, sem)]

        pending = stage(0)
        out_pending = [None, None]
        for j in range(poses_per_tile):
            buf = j & 1
            cx, cy, cz, off_v, bt_v, ic_v, out_v, _, sout = bufs[buf]
            comp_refs = (cx, cy, cz)
            for h in pending:
                h.wait()
            if j + 1 < poses_per_tile:
                pending = stage(j + 1)
            if out_pending[buf] is not None:
                out_pending[buf].wait()

            def chase(i, _carry):
                sl = pl.ds(i * _LANES, _LANES)
                bvec = i * _LANES + lax.iota(jnp.int32, _LANES)
                zero = jnp.zeros((_LANES,), jnp.int32)
                bt = bt_v[sl]
                conn1 = plsc.load_gather(tbl_v, [bt])
                b1 = plsc.load_gather(tbl_v, [128 + bt * MAX_CONNS + conn1])
                c0 = plsc.load_gather(ic_v, [conn1, zero, bvec])
                c1 = plsc.load_gather(ic_v, [conn1, zero + 1, bvec])
                nbr = lax.bitwise_and(c0, MAX_BLOCKS - 1)
                nconn = lax.rem(c1, MAX_CONNS)
                nbt = plsc.load_gather(bt_v, [nbr])
                off2 = plsc.load_gather(off_v, [nbr])
                b2 = plsc.load_gather(tbl_v, [128 + nbt * MAX_CONNS + nconn])
                s1_v[sl] = off_v[sl] + b1
                s2_v[sl] = off2 + b2
                out_v[18, sl] = jnp.where(nbr > bvec, jnp.full((_LANES,), 1.0, jnp.float32),
                                          jnp.full((_LANES,), 0.0, jnp.float32))
                return _carry

            lax.fori_loop(0, _CHUNKS, chase, jnp.int32(0), unroll=8)

            def fetch(i, _carry):
                sl = pl.ds(i * _LANES, _LANES)
                s1 = s1_v[sl]
                s2 = s2_v[sl]
                for atom in range(3):
                    for c in range(3):
                        out_v[atom * 3 + c, sl] = plsc.load_gather(
                            comp_refs[c], [s1 + atom])
                        out_v[9 + atom * 3 + c, sl] = plsc.load_gather(
                            comp_refs[c], [s2 + atom])
                return _carry

            lax.fori_loop(0, _CHUNKS, fetch, jnp.int32(0), unroll=4)
            p = wid * poses_per_tile + j
            out_pending[buf] = pltpu.async_copy(out_v, out_hbm.at[:, p], sout)
        for h in out_pending:
            if h is not None:
                h.wait()

    return k(coords_t, offs, bts, iconns_t, tbl)


def _tc_body(gp_ref, packed_ref, out_ref):
    p = [gp_ref[0, i] for i in range(34)]

    def row(r):
        return packed_ref[r]

    SG1 = [row(0), row(1), row(2)]
    CB1 = [row(3), row(4), row(5)]
    CA1 = [row(6), row(7), row(8)]
    SG2 = [row(9), row(10), row(11)]
    CB2 = [row(12), row(13), row(14)]
    CA2 = [row(15), row(16), row(17)]
    mask = row(18)

    def sub(u, v):
        return [u[0] - v[0], u[1] - v[1], u[2] - v[2]]

    def dot(u, v):
        return u[0] * v[0] + u[1] * v[1] + u[2] * v[2]

    def cross(u, v):
        return [u[1] * v[2] - u[2] * v[1],
                u[2] * v[0] - u[0] * v[2],
                u[0] * v[1] - u[1] * v[0]]

    def angle_cs(a, b, c):
        u = sub(a, b)
        v = sub(c, b)
        un = jnp.sqrt(dot(u, u) + 1e-12)
        vn = jnp.sqrt(dot(v, v) + 1e-12)
        cosv = jnp.clip(dot(u, v) / (un * vn), -1.0 + 1e-6, 1.0 - 1e-6)
        return cosv, jnp.sqrt(1.0 - cosv * cosv)

    def dihedral_cs(q0, q1, q2, q3):
        b0 = sub(q0, q1)
        b1 = sub(q2, q1)
        b2 = sub(q3, q2)
        inv = 1.0 / (jnp.sqrt(dot(b1, b1)) + 1e-8)
        b1n = [b1[0] * inv, b1[1] * inv, b1[2] * inv]
        d0 = dot(b0, b1n)
        d2 = dot(b2, b1n)
        v = [b0[0] - d0 * b1n[0], b0[1] - d0 * b1n[1], b0[2] - d0 * b1n[2]]
        w = [b2[0] - d2 * b1n[0], b2[1] - d2 * b1n[1], b2[2] - d2 * b1n[2]]
        x = dot(v, w) + 1e-12
        y = dot(cross(b1n, v), w)
        rinv = 1.0 / jnp.maximum(jnp.sqrt(x * x + y * y), 1e-30)
        return x * rinv, y * rinv

    dvec = sub(SG1, SG2)
    d = jnp.sqrt(dot(dvec, dvec) + 1e-12)
    ang1 = angle_cs(CB1, SG1, SG2)
    ang2 = angle_cs(CB2, SG2, SG1)
    chi_ss = dihedral_cs(CB1, SG1, SG2, CB2)
    chi1 = dihedral_cs(CA1, CB1, SG1, SG2)
    chi2 = dihedral_cs(CA2, CB2, SG2, SG1)

    scale = p[1] + 1e-6
    z = (d - p[0]) / scale
    log_pdf = -0.5 * z * z - 0.5 * jnp.log(2.0 * jnp.pi)
    x = p[2] * z
    xs = x * 0.7071067811865476
    pos = jnp.maximum(0.5 * (1.0 + lax.erf(xs)), 1e-38)
    w = -xs
    t = w
    for cf_k in range(6, 0, -1):
        t = w + (0.5 * cf_k) / t
    neg_lc = -w * w - jnp.log(t) - 1.2655121234846454
    log_cdf = jnp.where(x < -3.0, neg_lc, jnp.log(pos))
    score_d = -(jnp.log(2.0) + log_pdf + log_cdf - p[33])

    def vm(a_cs, logA, kappa, cos_mu, sin_mu):
        return logA + kappa * (a_cs[0] * cos_mu + a_cs[1] * sin_mu)

    score_a = -(vm(ang1, p[3], p[4], p[21], p[22]) +
                vm(ang2, p[3], p[4], p[21], p[22]))
    score_ss = -jnp.logaddexp(vm(chi_ss, p[6], p[7], p[23], p[24]),
                              vm(chi_ss, p[9], p[10], p[25], p[26]))

    def cs(a_cs):
        return -jnp.logaddexp(
            jnp.logaddexp(vm(a_cs, p[12], p[14], p[27], p[28]),
                          vm(a_cs, p[15], p[17], p[29], p[30])),
            vm(a_cs, p[18], p[20], p[31], p[32]))

    total = score_d + score_a + score_ss + cs(chi1) + cs(chi2)
    per_pose = jnp.sum(total * mask, axis=1)
    out_ref[...] = per_pose[:, None]


def _tc_score(packed, gp_raw, interpret=False):
    mus = jnp.stack([gp_raw[0, 5], gp_raw[0, 8], gp_raw[0, 11],
                     gp_raw[0, 13], gp_raw[0, 16], gp_raw[0, 19]])
    trig = jnp.stack([jnp.cos(mus), jnp.sin(mus)], axis=1).reshape(12)
    log_scale = jnp.log(gp_raw[0, 1] + 1e-6)
    gp = jnp.concatenate([gp_raw, trig[None, :], log_scale[None, None]], axis=1)
    PB = 8
    n_poses = packed.shape[1]
    grid = (n_poses // PB,)
    out = pl.pallas_call(
        _tc_body,
        grid=grid,
        in_specs=[
            pl.BlockSpec(memory_space=pltpu.SMEM),
            pl.BlockSpec((_PACK_ROWS, PB, MAX_BLOCKS), lambda g: (0, g, 0)),
        ],
        out_specs=pl.BlockSpec((PB, 1), lambda g: (g, 0)),
        out_shape=jax.ShapeDtypeStruct((n_poses, 1), jnp.float32),
        interpret=interpret,
    )(gp, packed)
    return out


def kernel(coords, pose_stack_block_coord_offset, pose_stack_block_types,
           pose_stack_inter_block_connections, bt_disulfide_conns,
           bt_atom_downstream_of_conn, global_params):
    coords_t = jnp.transpose(coords, (2, 0, 1))
    offs = pose_stack_block_coord_offset.astype(jnp.int32)
    bts = pose_stack_block_types.astype(jnp.int32)
    iconns_t = jnp.transpose(
        pose_stack_inter_block_connections.astype(jnp.int32), (0, 2, 3, 1))
    dconns = bt_disulfide_conns.astype(jnp.int32)
    dsbase = bt_atom_downstream_of_conn[:, :, 0].reshape(
        N_BT * MAX_CONNS).astype(jnp.int32)
    tbl = jnp.concatenate([dconns, jnp.zeros((28,), jnp.int32), dsbase,
                           jnp.zeros((84,), jnp.int32)])
    packed = _sc_gather(coords_t, offs, bts, iconns_t, tbl, 0, N_POSES)
    return _tc_score(packed, global_params).reshape(1, N_POSES)


# speedup vs baseline: 1.1327x; 1.1327x over previous
import functools

import jax
import jax.numpy as jnp
from jax import lax
from jax.experimental import pallas as pl
from jax.experimental.pallas import tpu as pltpu
from jax.experimental.pallas import tpu_sc as plsc

N_POSES = 128
MAX_BLOCKS = 512
ATOMS_PER_BLOCK = 16
MAX_ATOMS = MAX_BLOCKS * ATOMS_PER_BLOCK
N_BT = 100
MAX_CONNS = 3

_N_TILES = 32
_POSES_PER_TILE = N_POSES // _N_TILES
_LANES = 16
_CHUNKS = MAX_BLOCKS // _LANES
_PACK_ROWS = 19


def _sc_gather(coords_t, offs, bts, iconns_t, tbl, pose_base, n_poses):
    poses_per_tile = n_poses // _N_TILES
    mesh = plsc.VectorSubcoreMesh(core_axis_name="c", subcore_axis_name="s")

    @functools.partial(
        pl.kernel,
        mesh=mesh,
        compiler_params=pltpu.CompilerParams(needs_layout_passes=False),
        out_type=jax.ShapeDtypeStruct((_PACK_ROWS, n_poses, MAX_BLOCKS), jnp.float32),
        scratch_types=[
            pltpu.VMEM((MAX_ATOMS,), jnp.float32),
            pltpu.VMEM((MAX_ATOMS,), jnp.float32),
            pltpu.VMEM((MAX_ATOMS,), jnp.float32),
            pltpu.VMEM((MAX_ATOMS,), jnp.float32),
            pltpu.VMEM((MAX_ATOMS,), jnp.float32),
            pltpu.VMEM((MAX_ATOMS,), jnp.float32),
            pltpu.VMEM((MAX_BLOCKS,), jnp.int32),
            pltpu.VMEM((MAX_BLOCKS,), jnp.int32),
            pltpu.VMEM((MAX_BLOCKS,), jnp.int32),
            pltpu.VMEM((MAX_BLOCKS,), jnp.int32),
            pltpu.VMEM((MAX_CONNS, 2, MAX_BLOCKS), jnp.int32),
            pltpu.VMEM((MAX_CONNS, 2, MAX_BLOCKS), jnp.int32),
            pltpu.VMEM((_PACK_ROWS, MAX_BLOCKS), jnp.float32),
            pltpu.VMEM((_PACK_ROWS, MAX_BLOCKS), jnp.float32),
            pltpu.VMEM((512,), jnp.int32),
            pltpu.SemaphoreType.DMA,
            pltpu.SemaphoreType.DMA,
            pltpu.SemaphoreType.DMA,
            pltpu.SemaphoreType.DMA,
        ],
    )
    def k(coords_hbm, offs_hbm, bts_hbm, iconns_hbm, tbl_hbm,
          out_hbm, cx0, cy0, cz0, cx1, cy1, cz1, offs0, bts0, offs1, bts1,
          ic0, ic1, o0v, o1v, tbl_v, sin0, sin1, sout0, sout1):
        wid = lax.axis_index("s") * 2 + lax.axis_index("c")
        bufs = [(cx0, cy0, cz0, offs0, bts0, ic0, o0v, sin0, sout0),
                (cx1, cy1, cz1, offs1, bts1, ic1, o1v, sin1, sout1)]
        pltpu.sync_copy(tbl_hbm, tbl_v)

        def stage(j):
            cx, cy, cz, off_v, bt_v, ic_v, _, sem, _2 = bufs[j & 1]
            p = pose_base + wid * poses_per_tile + j
            return [pltpu.async_copy(coords_hbm.at[0, p], cx, sem),
                    pltpu.async_copy(coords_hbm.at[1, p], cy, sem),
                    pltpu.async_copy(coords_hbm.at[2, p], cz, sem),
                    pltpu.async_copy(offs_hbm.at[p], off_v, sem),
                    pltpu.async_copy(bts_hbm.at[p], bt_v, sem),
                    pltpu.async_copy(iconns_hbm.at[p], ic_v, sem)]

        pending = stage(0)
        out_pending = [None, None]
        for j in range(poses_per_tile):
            buf = j & 1
            cx, cy, cz, off_v, bt_v, ic_v, out_v, _, sout = bufs[buf]
            comp_refs = (cx, cy, cz)
            for h in pending:
                h.wait()
            if j + 1 < poses_per_tile:
                pending = stage(j + 1)
            if out_pending[buf] is not None:
                out_pending[buf].wait()

            def chunk(i, _carry):
                sl = pl.ds(i * _LANES, _LANES)
                bvec = i * _LANES + lax.iota(jnp.int32, _LANES)
                zero = jnp.zeros((_LANES,), jnp.int32)
                bt = bt_v[sl]
                cb = plsc.load_gather(tbl_v, [bt])
                conn1 = lax.shift_right_logical(cb, 4)
                b1 = lax.bitwise_and(cb, 15)
                c0 = plsc.load_gather(ic_v, [conn1, zero, bvec])
                c1 = plsc.load_gather(ic_v, [conn1, zero + 1, bvec])
                nbr = lax.bitwise_and(c0, MAX_BLOCKS - 1)
                nconn = lax.rem(c1, MAX_CONNS)
                nbt = plsc.load_gather(bt_v, [nbr])
                off1 = off_v[sl]
                off2 = plsc.load_gather(off_v, [nbr])
                b2 = plsc.load_gather(tbl_v, [128 + nbt * MAX_CONNS + nconn])
                s1 = off1 + b1
                s2 = off2 + b2
                for atom in range(3):
                    for c in range(3):
                        out_v[atom * 3 + c, sl] = plsc.load_gather(
                            comp_refs[c], [s1 + atom])
                        out_v[9 + atom * 3 + c, sl] = plsc.load_gather(
                            comp_refs[c], [s2 + atom])
                out_v[18, sl] = jnp.where(nbr > bvec, jnp.full((_LANES,), 1.0, jnp.float32),
                                          jnp.full((_LANES,), 0.0, jnp.float32))
                return _carry

            lax.fori_loop(0, _CHUNKS, chunk, jnp.int32(0), unroll=4)
            p = wid * poses_per_tile + j
            out_pending[buf] = pltpu.async_copy(out_v, out_hbm.at[:, p], sout)
        for h in out_pending:
            if h is not None:
                h.wait()

    return k(coords_t, offs, bts, iconns_t, tbl)


def _tc_body(gp_ref, packed_ref, out_ref):
    p = [gp_ref[0, i] for i in range(34)]

    def row(r):
        return packed_ref[r]

    SG1 = [row(0), row(1), row(2)]
    CB1 = [row(3), row(4), row(5)]
    CA1 = [row(6), row(7), row(8)]
    SG2 = [row(9), row(10), row(11)]
    CB2 = [row(12), row(13), row(14)]
    CA2 = [row(15), row(16), row(17)]
    mask = row(18)

    def sub(u, v):
        return [u[0] - v[0], u[1] - v[1], u[2] - v[2]]

    def dot(u, v):
        return u[0] * v[0] + u[1] * v[1] + u[2] * v[2]

    def cross(u, v):
        return [u[1] * v[2] - u[2] * v[1],
                u[2] * v[0] - u[0] * v[2],
                u[0] * v[1] - u[1] * v[0]]

    def angle_cs(a, b, c):
        u = sub(a, b)
        v = sub(c, b)
        un = jnp.sqrt(dot(u, u) + 1e-12)
        vn = jnp.sqrt(dot(v, v) + 1e-12)
        cosv = jnp.clip(dot(u, v) / (un * vn), -1.0 + 1e-6, 1.0 - 1e-6)
        return cosv, jnp.sqrt(1.0 - cosv * cosv)

    def dihedral_cs(q0, q1, q2, q3):
        b0 = sub(q0, q1)
        b1 = sub(q2, q1)
        b2 = sub(q3, q2)
        inv = 1.0 / (jnp.sqrt(dot(b1, b1)) + 1e-8)
        b1n = [b1[0] * inv, b1[1] * inv, b1[2] * inv]
        d0 = dot(b0, b1n)
        d2 = dot(b2, b1n)
        v = [b0[0] - d0 * b1n[0], b0[1] - d0 * b1n[1], b0[2] - d0 * b1n[2]]
        w = [b2[0] - d2 * b1n[0], b2[1] - d2 * b1n[1], b2[2] - d2 * b1n[2]]
        x = dot(v, w) + 1e-12
        y = dot(cross(b1n, v), w)
        rinv = 1.0 / jnp.maximum(jnp.sqrt(x * x + y * y), 1e-30)
        return x * rinv, y * rinv

    dvec = sub(SG1, SG2)
    d = jnp.sqrt(dot(dvec, dvec) + 1e-12)
    ang1 = angle_cs(CB1, SG1, SG2)
    ang2 = angle_cs(CB2, SG2, SG1)
    chi_ss = dihedral_cs(CB1, SG1, SG2, CB2)
    chi1 = dihedral_cs(CA1, CB1, SG1, SG2)
    chi2 = dihedral_cs(CA2, CB2, SG2, SG1)

    scale = p[1] + 1e-6
    z = (d - p[0]) / scale
    log_pdf = -0.5 * z * z - 0.5 * jnp.log(2.0 * jnp.pi)
    x = p[2] * z
    xs = x * 0.7071067811865476
    pos = jnp.maximum(0.5 * (1.0 + lax.erf(xs)), 1e-38)
    w = -xs
    t = w
    for cf_k in range(6, 0, -1):
        t = w + (0.5 * cf_k) / t
    neg_lc = -w * w - jnp.log(t) - 1.2655121234846454
    log_cdf = jnp.where(x < -3.0, neg_lc, jnp.log(pos))
    score_d = -(jnp.log(2.0) + log_pdf + log_cdf - p[33])

    def vm(a_cs, logA, kappa, cos_mu, sin_mu):
        return logA + kappa * (a_cs[0] * cos_mu + a_cs[1] * sin_mu)

    score_a = -(vm(ang1, p[3], p[4], p[21], p[22]) +
                vm(ang2, p[3], p[4], p[21], p[22]))
    score_ss = -jnp.logaddexp(vm(chi_ss, p[6], p[7], p[23], p[24]),
                              vm(chi_ss, p[9], p[10], p[25], p[26]))

    def cs(a_cs):
        return -jnp.logaddexp(
            jnp.logaddexp(vm(a_cs, p[12], p[14], p[27], p[28]),
                          vm(a_cs, p[15], p[17], p[29], p[30])),
            vm(a_cs, p[18], p[20], p[31], p[32]))

    total = score_d + score_a + score_ss + cs(chi1) + cs(chi2)
    per_pose = jnp.sum(total * mask, axis=1)
    out_ref[...] = per_pose[:, None]


def _tc_score(packed, gp_raw, interpret=False):
    mus = jnp.stack([gp_raw[0, 5], gp_raw[0, 8], gp_raw[0, 11],
                     gp_raw[0, 13], gp_raw[0, 16], gp_raw[0, 19]])
    trig = jnp.stack([jnp.cos(mus), jnp.sin(mus)], axis=1).reshape(12)
    log_scale = jnp.log(gp_raw[0, 1] + 1e-6)
    gp = jnp.concatenate([gp_raw, trig[None, :], log_scale[None, None]], axis=1)
    PB = 16
    n_poses = packed.shape[1]
    grid = (n_poses // PB,)
    out = pl.pallas_call(
        _tc_body,
        grid=grid,
        in_specs=[
            pl.BlockSpec(memory_space=pltpu.SMEM),
            pl.BlockSpec((_PACK_ROWS, PB, MAX_BLOCKS), lambda g: (0, g, 0)),
        ],
        out_specs=pl.BlockSpec((PB, 1), lambda g: (g, 0)),
        out_shape=jax.ShapeDtypeStruct((n_poses, 1), jnp.float32),
        interpret=interpret,
    )(gp, packed)
    return out


def kernel(coords, pose_stack_block_coord_offset, pose_stack_block_types,
           pose_stack_inter_block_connections, bt_disulfide_conns,
           bt_atom_downstream_of_conn, global_params):
    coords_t = jnp.transpose(coords, (2, 0, 1))
    offs = pose_stack_block_coord_offset.astype(jnp.int32)
    bts = pose_stack_block_types.astype(jnp.int32)
    iconns_t = jnp.transpose(
        pose_stack_inter_block_connections.astype(jnp.int32), (0, 2, 3, 1))
    dconns = bt_disulfide_conns.astype(jnp.int32)
    dsbase = bt_atom_downstream_of_conn[:, :, 0].reshape(
        N_BT * MAX_CONNS).astype(jnp.int32)
    cb = dconns * 16 + dsbase[jnp.arange(N_BT, dtype=jnp.int32) * MAX_CONNS + dconns]
    tbl = jnp.concatenate([cb, jnp.zeros((28,), jnp.int32), dsbase,
                           jnp.zeros((84,), jnp.int32)])
    packed = _sc_gather(coords_t, offs, bts, iconns_t, tbl, 0, N_POSES)
    return _tc_score(packed, global_params).reshape(1, N_POSES)
